# baseline (speedup 1.0000x reference)
import functools

import jax
import jax.numpy as jnp
from jax import lax
from jax.experimental import pallas as pl
from jax.experimental.pallas import tpu as pltpu
from jax.experimental.pallas import tpu_sc as plsc

B = 4096
DENSE_DIM = 13
N_FIELDS = 26
VOCAB = 100000
EMBED_DIM = 32
BF = B * N_FIELDS
ROWS_PER_LINE = 128 // EMBED_DIM
TAB_LINES = N_FIELDS * VOCAB // ROWS_PER_LINE

_LANES = 16
_CHUNK = 128


def _sc_gather_make(num_workers: int, per_w: int):
  mesh = plsc.VectorSubcoreMesh(core_axis_name="c", subcore_axis_name="s")
  n_chunks = per_w // _CHUNK

  @functools.partial(
      pl.kernel,
      mesh=mesh,
      compiler_params=pltpu.CompilerParams(
          use_tc_tiling_on_sc=True, needs_layout_passes=False),
      out_type=jax.ShapeDtypeStruct((BF * EMBED_DIM,), jnp.float32),
      scratch_types=[
          pltpu.VMEM((per_w,), jnp.int32),
          pltpu.VMEM((per_w,), jnp.int32),
          pltpu.VMEM((2, _CHUNK, 128), jnp.float32),
          pltpu.VMEM((_CHUNK * EMBED_DIM,), jnp.float32),
          pltpu.SemaphoreType.DMA,
      ],
  )
  def gather_k(tab_hbm, sp_hbm, out_hbm, idx_v, g_v, buf_v, outc_v, sem):
    wid = lax.axis_index("s") * 2 + lax.axis_index("c")
    base = wid * per_w
    pltpu.sync_copy(sp_hbm.at[pl.ds(base, per_w)], idx_v)
    iota = lax.iota(jnp.int32, _LANES)
    def idx_body(k, carry):
      sl = pl.ds(k * _LANES, _LANES)
      pos = base + k * _LANES + iota
      flat = idx_v[sl] + lax.rem(pos, N_FIELDS) * VOCAB
      idx_v[sl] = lax.shift_right_logical(flat, 2)
      g_v[sl] = lax.bitwise_and(flat, 3) * EMBED_DIM
      return carry
    lax.fori_loop(0, per_w // _LANES, idx_body, 0)

    def fire(j, slot):
      sl = pl.ds(j * _CHUNK, _CHUNK)
      pltpu.async_copy(tab_hbm.at[idx_v.at[sl]], buf_v.at[slot], sem)

    def drain(j, slot):
      sl = pl.ds(j * _CHUNK, _CHUNK)
      pltpu.make_async_copy(tab_hbm.at[idx_v.at[sl]], buf_v.at[slot], sem).wait()

    fire(0, 0)
    def chunk_body(j, carry):
      slot = lax.rem(j, 2)
      drain(j, slot)
      @pl.when(j + 1 < n_chunks)
      def _():
        fire(j + 1, lax.rem(j + 1, 2))
      gsl = g_v.at[pl.ds(j * _CHUNK, _CHUNK)]
      bsl = buf_v.at[slot]
      def row_body(r, carry2):
        rvec = jnp.broadcast_to(r, (_LANES,)).astype(jnp.int32)
        gvec = plsc.load_gather(gsl, [rvec])
        for half in range(EMBED_DIM // _LANES):
          col = gvec + half * _LANES + iota
          vals = plsc.load_gather(bsl, [rvec, col])
          outc_v[pl.ds(r * EMBED_DIM + half * _LANES, _LANES)] = vals
        return carry2
      lax.fori_loop(0, _CHUNK, row_body, 0)
      pltpu.sync_copy(
          outc_v,
          out_hbm.at[pl.ds((base + j * _CHUNK) * EMBED_DIM, _CHUNK * EMBED_DIM)])
      return carry
    lax.fori_loop(0, n_chunks, chunk_body, 0)

  return gather_k


def _mlp_body(dense_ref, embs_ref, w1d_ref, w1e_ref, b1_ref, w2_ref, b2_ref,
              w3_ref, b3_ref, out_ref):
  x1 = (dense_ref[...] @ w1d_ref[...] + embs_ref[...] @ w1e_ref[...]
        + b1_ref[...])
  h1 = jnp.maximum(x1, 0.0)
  h2 = jnp.maximum(h1 @ w2_ref[...] + b2_ref[...], 0.0)
  o = h2 @ w3_ref[...] + b3_ref[...]
  out_ref[...] = jax.nn.sigmoid(o)


def kernel(dense, sparse, tables, W1, b1, W2, b2, W3, b3):
  tab_wide = tables.reshape(TAB_LINES, 128)
  sp_flat = sparse.reshape(BF)

  info = plsc.get_sparse_core_info()
  nw = info.num_cores * info.num_subcores
  per_w = BF // nw
  embs = _sc_gather_make(nw, per_w)(tab_wide, sp_flat)
  embs = embs.reshape(B, N_FIELDS * EMBED_DIM)

  w1d = W1[:DENSE_DIM]
  w1e = W1[DENSE_DIM:]
  bs = 512
  grid = (B // bs,)
  full = lambda shape: pl.BlockSpec(shape, lambda i: (0, 0))
  out = pl.pallas_call(
      _mlp_body,
      grid=grid,
      in_specs=[
          pl.BlockSpec((bs, DENSE_DIM), lambda i: (i, 0)),
          pl.BlockSpec((bs, N_FIELDS * EMBED_DIM), lambda i: (i, 0)),
          full(w1d.shape),
          full(w1e.shape),
          pl.BlockSpec((1, 128), lambda i: (0, 0)),
          full(W2.shape),
          pl.BlockSpec((1, 64), lambda i: (0, 0)),
          full(W3.shape),
          pl.BlockSpec((1, 1), lambda i: (0, 0)),
      ],
      out_specs=pl.BlockSpec((bs, 1), lambda i: (i, 0)),
      out_shape=jax.ShapeDtypeStruct((B, 1), jnp.float32),
  )(dense, embs, w1d, w1e, b1.reshape(1, 128), W2, b2.reshape(1, 64), W3,
    b3.reshape(1, 1))
  return out.reshape(B)


# speedup vs baseline: 1.9081x; 1.9081x over previous
import functools

import jax
import jax.numpy as jnp
from jax import lax
from jax.experimental import pallas as pl
from jax.experimental.pallas import tpu as pltpu
from jax.experimental.pallas import tpu_sc as plsc

B = 4096
DENSE_DIM = 13
N_FIELDS = 26
VOCAB = 100000
EMBED_DIM = 32
BF = B * N_FIELDS
TAB_WORDS = N_FIELDS * VOCAB * EMBED_DIM

_LANES = 16
_CHUNK = 128
_HALF_ITEMS = 1664
_HALF_WORDS = _HALF_ITEMS * EMBED_DIM


def _sc_gather_make(num_workers: int, per_w: int):
  mesh = plsc.VectorSubcoreMesh(core_axis_name="c", subcore_axis_name="s")

  @functools.partial(
      pl.kernel,
      mesh=mesh,
      compiler_params=pltpu.CompilerParams(needs_layout_passes=False),
      out_type=jax.ShapeDtypeStruct((BF * EMBED_DIM,), jnp.float32),
      scratch_types=[
          pltpu.VMEM((per_w,), jnp.int32),
          pltpu.VMEM((_HALF_WORDS,), jnp.int32),
          pltpu.VMEM((_HALF_WORDS,), jnp.float32),
          pltpu.SemaphoreType.DMA,
      ],
  )
  def gather_k(tab_hbm, sp_hbm, out_hbm, base_v, widx_v, data_v, sem):
    wid = lax.axis_index("s") * 2 + lax.axis_index("c")
    base = wid * per_w
    pltpu.sync_copy(sp_hbm.at[pl.ds(base, per_w)], base_v)
    iota = lax.iota(jnp.int32, _LANES)
    def base_body(k, carry):
      sl = pl.ds(k * _LANES, _LANES)
      pos = base + k * _LANES + iota
      base_v[sl] = base_v[sl] + lax.rem(pos, N_FIELDS) * (VOCAB * EMBED_DIM)
      return carry
    lax.fori_loop(0, per_w // _LANES, base_body, 0)

    c_lo = iota * VOCAB
    c_hi = (iota + _LANES) * VOCAB
    n_chunks = _HALF_WORDS // _CHUNK
    for h in range(per_w // _HALF_ITEMS):
      hoff = h * _HALF_ITEMS
      def expand(m, carry):
        bvec = plsc.load_gather(
            base_v, [jnp.broadcast_to(hoff + m, (_LANES,)).astype(jnp.int32)])
        widx_v[pl.ds(m * EMBED_DIM, _LANES)] = bvec + c_lo
        widx_v[pl.ds(m * EMBED_DIM + _LANES, _LANES)] = bvec + c_hi
        return carry
      lax.fori_loop(0, _HALF_ITEMS, expand, 0)
      def fire(j, carry):
        sl = pl.ds(j * _CHUNK, _CHUNK)
        pltpu.async_copy(tab_hbm.at[widx_v.at[sl]], data_v.at[sl], sem)
        return carry
      lax.fori_loop(0, n_chunks, fire, 0)
      def drain(j, carry):
        sl = pl.ds(j * _CHUNK, _CHUNK)
        pltpu.make_async_copy(
            tab_hbm.at[widx_v.at[sl]], data_v.at[sl], sem).wait()
        return carry
      lax.fori_loop(0, n_chunks, drain, 0)
      pltpu.sync_copy(
          data_v,
          out_hbm.at[pl.ds((base + hoff) * EMBED_DIM, _HALF_WORDS)])

  return gather_k


def _mlp_body(dense_ref, embs_ref, w1d_ref, w1e_ref, b1_ref, w2_ref, b2_ref,
              w3_ref, b3_ref, out_ref):
  x1 = (dense_ref[...] @ w1d_ref[...] + embs_ref[...] @ w1e_ref[...]
        + b1_ref[...])
  h1 = jnp.maximum(x1, 0.0)
  h2 = jnp.maximum(h1 @ w2_ref[...] + b2_ref[...], 0.0)
  o = h2 @ w3_ref[...] + b3_ref[...]
  out_ref[...] = jax.nn.sigmoid(o)


def kernel(dense, sparse, tables, W1, b1, W2, b2, W3, b3):
  tab_flat = jnp.transpose(tables, (0, 2, 1)).reshape(TAB_WORDS)
  sp_flat = sparse.reshape(BF)

  info = plsc.get_sparse_core_info()
  nw = info.num_cores * info.num_subcores
  per_w = BF // nw
  embs = _sc_gather_make(nw, per_w)(tab_flat, sp_flat)
  embs = embs.reshape(B, N_FIELDS * EMBED_DIM)

  w1d = W1[:DENSE_DIM]
  w1e = W1[DENSE_DIM:]
  bs = 512
  grid = (B // bs,)
  full = lambda shape: pl.BlockSpec(shape, lambda i: (0, 0))
  out = pl.pallas_call(
      _mlp_body,
      grid=grid,
      in_specs=[
          pl.BlockSpec((bs, DENSE_DIM), lambda i: (i, 0)),
          pl.BlockSpec((bs, N_FIELDS * EMBED_DIM), lambda i: (i, 0)),
          full(w1d.shape),
          full(w1e.shape),
          pl.BlockSpec((1, 128), lambda i: (0, 0)),
          full(W2.shape),
          pl.BlockSpec((1, 64), lambda i: (0, 0)),
          full(W3.shape),
          pl.BlockSpec((1, 1), lambda i: (0, 0)),
      ],
      out_specs=pl.BlockSpec((bs, 1), lambda i: (i, 0)),
      out_shape=jax.ShapeDtypeStruct((B, 1), jnp.float32),
  )(dense, embs, w1d, w1e, b1.reshape(1, 128), W2, b2.reshape(1, 64), W3,
    b3.reshape(1, 1))
  return out.reshape(B)
